# Initial kernel scaffold; baseline (speedup 1.0000x reference)
import jax
import jax.numpy as jnp
from jax.experimental import pallas as pl


def kernel(H, D_gt, P_gt, E_gt, dp_w1, dp_b1, dp_w2, dp_b2, dp_wl, dp_bl, pp_w1, pp_b1, pp_w2, pp_b2, pp_wl, pp_bl, ep_w1, ep_b1, ep_w2, ep_b2, ep_wl, ep_bl, ppj_w, ppj_b, epj_w, epj_b):
    raise NotImplementedError("write your pallas kernel here")


# speedup vs baseline: 22.9910x; 22.9910x over previous
import functools

import jax
import jax.numpy as jnp
from jax.experimental import pallas as pl
from jax.experimental.pallas import tpu as pltpu

B, S, D_MODEL = 16, 512, 256
MAX_T = 2048
F = 256


def _conv3(x, w_ref, b):
    cin = x.shape[1]
    zrow = jnp.zeros((1, cin), x.dtype)
    xm = jnp.concatenate([zrow, x[:-1]], axis=0)
    xp = jnp.concatenate([x[1:], zrow], axis=0)
    y = jnp.dot(xm, w_ref[0], preferred_element_type=jnp.float32)
    y = y + jnp.dot(x, w_ref[1], preferred_element_type=jnp.float32)
    y = y + jnp.dot(xp, w_ref[2], preferred_element_type=jnp.float32)
    return y + b


def _dp_body(h_ref, w1_ref, b1_ref, w2_ref, b2_ref, wl_ref, bl_ref, out_ref):
    x = h_ref[0]
    h1 = jax.nn.relu(_conv3(x, w1_ref, b1_ref[...]))
    h2 = jax.nn.relu(_conv3(h1, w2_ref, b2_ref[...]))
    out_ref[0] = jnp.dot(h2, wl_ref[...], preferred_element_type=jnp.float32) + bl_ref[...]


def _expand_body(d_ref, h_ref, out_ref):
    d = jnp.maximum(d_ref[0], 0).astype(jnp.float32)
    ii = jax.lax.broadcasted_iota(jnp.int32, (S, S), 0)
    jj = jax.lax.broadcasted_iota(jnp.int32, (S, S), 1)
    tri = (ii <= jj).astype(jnp.float32)
    c = jnp.dot(d, tri, preferred_element_type=jnp.float32)
    cm1 = c - d
    t = jax.lax.broadcasted_iota(jnp.int32, (MAX_T, S), 0).astype(jnp.float32)
    oh = jnp.where((t < c) & (t >= cm1), 1.0, 0.0)
    out_ref[0] = jnp.dot(oh, h_ref[0], preferred_element_type=jnp.float32)


def _fused_body(hexp_ref, pgt_ref, egt_ref,
                wj1_ref, bj1_ref, pw2_ref, pb2_ref, ew2_ref, eb2_ref,
                pwl_ref, pbl_ref, ewl_ref, ebl_ref,
                ppjw_ref, ppjb_ref, epjw_ref, epjb_ref,
                ha_ref, pp_ref, ep_ref):
    x = hexp_ref[0]
    p = pgt_ref[0]
    e = egt_ref[0]
    ha_ref[0] = (x + p * ppjw_ref[...] + ppjb_ref[...]
                 + e * epjw_ref[...] + epjb_ref[...])
    h1 = jax.nn.relu(_conv3(x, wj1_ref, bj1_ref[...]))
    h2p = jax.nn.relu(_conv3(h1[:, :F], pw2_ref, pb2_ref[...]))
    h2e = jax.nn.relu(_conv3(h1[:, F:], ew2_ref, eb2_ref[...]))
    pp_ref[0] = jnp.dot(h2p, pwl_ref[...], preferred_element_type=jnp.float32) + pbl_ref[...]
    ep_ref[0] = jnp.dot(h2e, ewl_ref[...], preferred_element_type=jnp.float32) + ebl_ref[...]


def _full(bs):
    n = len(bs)
    return pl.BlockSpec(bs, lambda b: (b,) + (0,) * (n - 1))


def _rep(bs):
    n = len(bs)
    return pl.BlockSpec(bs, lambda b: (0,) * n)


def kernel(H, D_gt, P_gt, E_gt, dp_w1, dp_b1, dp_w2, dp_b2, dp_wl, dp_bl,
           pp_w1, pp_b1, pp_w2, pp_b2, pp_wl, pp_bl,
           ep_w1, ep_b1, ep_w2, ep_b2, ep_wl, ep_bl,
           ppj_w, ppj_b, epj_w, epj_b):
    f32 = jnp.float32
    def taps(w):
        return jnp.transpose(w, (2, 1, 0))
    dp_w1t, dp_w2t = taps(dp_w1), taps(dp_w2)
    wj1 = jnp.concatenate([taps(pp_w1), taps(ep_w1)], axis=2)
    bj1 = jnp.concatenate([pp_b1, ep_b1])[None, :]
    pw2t, ew2t = taps(pp_w2), taps(ep_w2)

    d_pred = pl.pallas_call(
        _dp_body,
        grid=(B,),
        in_specs=[_full((1, S, D_MODEL)), _rep((3, D_MODEL, F)), _rep((1, F)),
                  _rep((3, F, F)), _rep((1, F)), _rep((F, 1)), _rep((1, 1))],
        out_specs=_full((1, S, 1)),
        out_shape=jax.ShapeDtypeStruct((B, S, 1), f32),
    )(H, dp_w1t, dp_b1[None, :], dp_w2t, dp_b2[None, :], dp_wl, dp_bl[None, :])

    h_exp = pl.pallas_call(
        _expand_body,
        grid=(B,),
        in_specs=[_full((1, 1, S)), _full((1, S, D_MODEL))],
        out_specs=_full((1, MAX_T, D_MODEL)),
        out_shape=jax.ShapeDtypeStruct((B, MAX_T, D_MODEL), f32),
    )(D_gt.reshape(B, 1, S), H)

    ha, ppred, epred = pl.pallas_call(
        _fused_body,
        grid=(B,),
        in_specs=[_full((1, MAX_T, D_MODEL)), _full((1, MAX_T, 1)), _full((1, MAX_T, 1)),
                  _rep((3, D_MODEL, 2 * F)), _rep((1, 2 * F)),
                  _rep((3, F, F)), _rep((1, F)), _rep((3, F, F)), _rep((1, F)),
                  _rep((F, 1)), _rep((1, 1)), _rep((F, 1)), _rep((1, 1)),
                  _rep((1, D_MODEL)), _rep((1, D_MODEL)), _rep((1, D_MODEL)), _rep((1, D_MODEL))],
        out_specs=[_full((1, MAX_T, D_MODEL)), _full((1, MAX_T, 1)), _full((1, MAX_T, 1))],
        out_shape=[jax.ShapeDtypeStruct((B, MAX_T, D_MODEL), f32),
                   jax.ShapeDtypeStruct((B, MAX_T, 1), f32),
                   jax.ShapeDtypeStruct((B, MAX_T, 1), f32)],
    )(h_exp, P_gt[..., None], E_gt[..., None],
      wj1, bj1, pw2t, pp_b2[None, :], ew2t, ep_b2[None, :],
      pp_wl, pp_bl[None, :], ep_wl, ep_bl[None, :],
      ppj_w[None, :], ppj_b[None, :], epj_w[None, :], epj_b[None, :])

    return (ha, d_pred[..., 0], ppred[..., 0], epred[..., 0])
